# Initial kernel scaffold; baseline (speedup 1.0000x reference)
import jax
import jax.numpy as jnp
from jax.experimental import pallas as pl


def kernel(moe_inp, gate_w, gate_b, w1, b1, w2, b2):
    raise NotImplementedError("write your pallas kernel here")


# speedup vs baseline: 1.1274x; 1.1274x over previous
import jax
import jax.numpy as jnp
from jax.experimental import pallas as pl
from jax.experimental.pallas import tpu as pltpu

NE = 8
DM = 1024
DF = 4096
TK = 2
NT = 2048
EPAD = 128

BLK = 256
NPAIR = NT * TK
TG = NPAIR // BLK + NE
PROWS = TG * BLK


def _gate_kernel(x_ref, gw_ref, gb_ref, o_ref):
    logits = jax.lax.dot_general(
        x_ref[...], gw_ref[...], (((1,), (1,)), ((), ())),
        preferred_element_type=jnp.float32)
    logits = logits + gb_ref[...]
    lanes = jax.lax.broadcasted_iota(jnp.int32, (NT, EPAD), 1)
    m1 = jnp.max(logits, axis=1, keepdims=True)
    i1 = jnp.min(jnp.where(logits == m1, lanes, EPAD), axis=1, keepdims=True)
    l2 = jnp.where(lanes == i1, -jnp.inf, logits)
    m2 = jnp.max(l2, axis=1, keepdims=True)
    i2 = jnp.min(jnp.where(l2 == m2, lanes, EPAD), axis=1, keepdims=True)
    s1 = 1.0 / (1.0 + jnp.exp(m2 - m1))
    s2 = 1.0 - s1
    out = (jnp.where(lanes == 0, i1.astype(jnp.float32), 0.0)
           + jnp.where(lanes == 1, i2.astype(jnp.float32), 0.0)
           + jnp.where(lanes == 2, s1, 0.0)
           + jnp.where(lanes == 3, s2, 0.0))
    o_ref[...] = out


def _ffn_kernel(emap_ref, valid_ref, x_ref, w1_ref, b1_ref, w2_ref, b2_ref,
                o_ref):
    t = pl.program_id(0)

    @pl.when(valid_ref[t] != 0)
    def _():
        x = x_ref[...]
        h = jax.lax.dot_general(x, w1_ref[0], (((1,), (1,)), ((), ())),
                                preferred_element_type=jnp.float32)
        h = jax.nn.gelu(h + b1_ref[0])
        y = jax.lax.dot_general(h.astype(jnp.bfloat16), w2_ref[0],
                                (((1,), (1,)), ((), ())),
                                preferred_element_type=jnp.float32)
        o_ref[...] = y + b2_ref[0]


def kernel(moe_inp, gate_w, gate_b, w1, b1, w2, b2):
    gw_pad = jnp.zeros((EPAD, DM), jnp.float32).at[:NE].set(gate_w)
    gb_pad = jnp.full((1, EPAD), -1e30, jnp.float32).at[0, :NE].set(gate_b)
    gate_out = pl.pallas_call(
        _gate_kernel,
        out_shape=jax.ShapeDtypeStruct((NT, EPAD), jnp.float32),
    )(moe_inp, gw_pad, gb_pad)
    top_i = gate_out[:, :TK].astype(jnp.int32)
    gate_s = gate_out[:, TK:2 * TK]

    eflat = top_i.reshape(-1)
    wflat = gate_s.reshape(-1)
    tok = jnp.arange(NPAIR, dtype=jnp.int32) // TK
    onehot = (eflat[:, None] == jnp.arange(NE, dtype=jnp.int32)[None, :])
    cum = jnp.cumsum(onehot.astype(jnp.int32), axis=0)
    rank = jnp.sum(jnp.where(onehot, cum, 0), axis=1) - 1
    counts = cum[-1]
    nblk = (counts + BLK - 1) // BLK
    blk_end = jnp.cumsum(nblk)
    astart = (blk_end - nblk) * BLK
    pos = astart[eflat] + rank
    gidx = jnp.zeros((PROWS,), jnp.int32).at[pos].set(tok)
    x_pad = jnp.take(moe_inp, gidx, axis=0).astype(jnp.bfloat16)
    tgrid = jnp.arange(TG, dtype=jnp.int32)
    emap = jnp.minimum(jnp.searchsorted(blk_end, tgrid, side='right'),
                       NE - 1).astype(jnp.int32)
    valid = (tgrid < blk_end[-1]).astype(jnp.int32)

    grid_spec = pltpu.PrefetchScalarGridSpec(
        num_scalar_prefetch=2,
        grid=(TG,),
        in_specs=[
            pl.BlockSpec((BLK, DM), lambda t, em, va: (t, 0)),
            pl.BlockSpec((1, DF, DM), lambda t, em, va: (em[t], 0, 0)),
            pl.BlockSpec((1, 1, DF), lambda t, em, va: (em[t], 0, 0)),
            pl.BlockSpec((1, DM, DF), lambda t, em, va: (em[t], 0, 0)),
            pl.BlockSpec((1, 1, DM), lambda t, em, va: (em[t], 0, 0)),
        ],
        out_specs=pl.BlockSpec((BLK, DM), lambda t, em, va: (t, 0)),
    )
    y_pad = pl.pallas_call(
        _ffn_kernel,
        grid_spec=grid_spec,
        out_shape=jax.ShapeDtypeStruct((PROWS, DM), jnp.float32),
    )(emap, valid, x_pad, w1.astype(jnp.bfloat16), b1.reshape(NE, 1, DF),
      w2.astype(jnp.bfloat16), b2.reshape(NE, 1, DM))

    yg = jnp.take(y_pad, pos, axis=0)
    out = jnp.sum((wflat[:, None] * yg).reshape(NT, TK, DM), axis=1)
    return out


# speedup vs baseline: 1.2877x; 1.1422x over previous
import jax
import jax.numpy as jnp
from jax.experimental import pallas as pl
from jax.experimental.pallas import tpu as pltpu

NE = 8
DM = 1024
DF = 4096
TK = 2
NT = 2048
EPAD = 128

BLK = 512
NPAIR = NT * TK
TG = NPAIR // BLK + NE
PROWS = TG * BLK


def _gate_kernel(x_ref, gw_ref, gb_ref, o_ref):
    logits = jax.lax.dot_general(
        x_ref[...], gw_ref[...], (((1,), (1,)), ((), ())),
        preferred_element_type=jnp.float32)
    logits = logits + gb_ref[...]
    lanes = jax.lax.broadcasted_iota(jnp.int32, (NT, EPAD), 1)
    m1 = jnp.max(logits, axis=1, keepdims=True)
    i1 = jnp.min(jnp.where(logits == m1, lanes, EPAD), axis=1, keepdims=True)
    l2 = jnp.where(lanes == i1, -jnp.inf, logits)
    m2 = jnp.max(l2, axis=1, keepdims=True)
    i2 = jnp.min(jnp.where(l2 == m2, lanes, EPAD), axis=1, keepdims=True)
    s1 = 1.0 / (1.0 + jnp.exp(m2 - m1))
    s2 = 1.0 - s1
    out = (jnp.where(lanes == 0, i1.astype(jnp.float32), 0.0)
           + jnp.where(lanes == 1, i2.astype(jnp.float32), 0.0)
           + jnp.where(lanes == 2, s1, 0.0)
           + jnp.where(lanes == 3, s2, 0.0))
    o_ref[...] = out


HF = DF // 2


def _ffn_kernel(emap_ref, valid_ref, x_ref, w1_ref, b1_ref, w2_ref, b2_ref,
                wrow_ref, o_ref):
    f = pl.program_id(0)
    t = pl.program_id(1)

    @pl.when(valid_ref[t] != 0)
    def _():
        x = x_ref[...]
        h = jax.lax.dot_general(x, w1_ref[0], (((1,), (1,)), ((), ())),
                                preferred_element_type=jnp.float32)
        h = jax.nn.gelu(h + b1_ref[0])
        y = jax.lax.dot_general(h, w2_ref[0], (((1,), (1,)), ((), ())),
                                preferred_element_type=jnp.float32)
        y = y + jnp.where(f == 0, 1.0, 0.0) * b2_ref[0]
        o_ref[0] = y * wrow_ref[...]


def kernel(moe_inp, gate_w, gate_b, w1, b1, w2, b2):
    gw_pad = jnp.zeros((EPAD, DM), jnp.float32).at[:NE].set(gate_w)
    gb_pad = jnp.full((1, EPAD), -1e30, jnp.float32).at[0, :NE].set(gate_b)
    gate_out = pl.pallas_call(
        _gate_kernel,
        out_shape=jax.ShapeDtypeStruct((NT, EPAD), jnp.float32),
    )(moe_inp, gw_pad, gb_pad)
    top_i = gate_out[:, :TK].astype(jnp.int32)
    gate_s = gate_out[:, TK:2 * TK]

    eflat = top_i.reshape(-1)
    wflat = gate_s.reshape(-1)
    tok = jnp.arange(NPAIR, dtype=jnp.int32) // TK
    onehot = (eflat[:, None] == jnp.arange(NE, dtype=jnp.int32)[None, :])
    cum = jnp.cumsum(onehot.astype(jnp.int32), axis=0)
    rank = jnp.sum(jnp.where(onehot, cum, 0), axis=1) - 1
    counts = cum[-1]
    nblk = (counts + BLK - 1) // BLK
    blk_end = jnp.cumsum(nblk)
    astart = (blk_end - nblk) * BLK
    pos = astart[eflat] + rank
    gidx = jnp.zeros((PROWS,), jnp.int32).at[pos].set(tok)
    x_pad = jnp.take(moe_inp, gidx, axis=0)
    w_pad = jnp.zeros((PROWS, 1), jnp.float32).at[pos, 0].set(wflat)
    tgrid = jnp.arange(TG, dtype=jnp.int32)
    emap = jnp.minimum(jnp.searchsorted(blk_end, tgrid, side='right'),
                       NE - 1).astype(jnp.int32)
    valid = (tgrid < blk_end[-1]).astype(jnp.int32)

    grid_spec = pltpu.PrefetchScalarGridSpec(
        num_scalar_prefetch=2,
        grid=(2, TG),
        in_specs=[
            pl.BlockSpec((BLK, DM), lambda f, t, em, va: (t, 0)),
            pl.BlockSpec((1, HF, DM), lambda f, t, em, va: (em[t], f, 0)),
            pl.BlockSpec((1, 1, HF), lambda f, t, em, va: (em[t], 0, f)),
            pl.BlockSpec((1, DM, HF), lambda f, t, em, va: (em[t], 0, f)),
            pl.BlockSpec((1, 1, DM), lambda f, t, em, va: (em[t], 0, 0)),
            pl.BlockSpec((BLK, 1), lambda f, t, em, va: (t, 0)),
        ],
        out_specs=pl.BlockSpec((1, BLK, DM), lambda f, t, em, va: (f, t, 0)),
    )
    y_slab = pl.pallas_call(
        _ffn_kernel,
        grid_spec=grid_spec,
        out_shape=jax.ShapeDtypeStruct((2, PROWS, DM), jnp.float32),
    )(emap, valid, x_pad, w1, b1.reshape(NE, 1, DF),
      w2, b2.reshape(NE, 1, DM), w_pad)

    ysum = y_slab[0] + y_slab[1]
    pp = pos.reshape(NT, TK)
    out = jnp.take(ysum, pp[:, 0], axis=0) + jnp.take(ysum, pp[:, 1], axis=0)
    return out


# speedup vs baseline: 1.2882x; 1.0004x over previous
import jax
import jax.numpy as jnp
from jax.experimental import pallas as pl
from jax.experimental.pallas import tpu as pltpu

NE = 8
DM = 1024
DF = 4096
TK = 2
NT = 2048
EPAD = 128

BLK = 512
NPAIR = NT * TK
TG = NPAIR // BLK + NE
PROWS = TG * BLK


def _gate_kernel(x_ref, gw_ref, gb_ref, o_ref):
    logits = jax.lax.dot_general(
        x_ref[...], gw_ref[...], (((1,), (1,)), ((), ())),
        preferred_element_type=jnp.float32)
    logits = logits + gb_ref[...]
    lanes = jax.lax.broadcasted_iota(jnp.int32, (NT, EPAD), 1)
    m1 = jnp.max(logits, axis=1, keepdims=True)
    i1 = jnp.min(jnp.where(logits == m1, lanes, EPAD), axis=1, keepdims=True)
    l2 = jnp.where(lanes == i1, -jnp.inf, logits)
    m2 = jnp.max(l2, axis=1, keepdims=True)
    i2 = jnp.min(jnp.where(l2 == m2, lanes, EPAD), axis=1, keepdims=True)
    s1 = 1.0 / (1.0 + jnp.exp(m2 - m1))
    s2 = 1.0 - s1
    out = (jnp.where(lanes == 0, i1.astype(jnp.float32), 0.0)
           + jnp.where(lanes == 1, i2.astype(jnp.float32), 0.0)
           + jnp.where(lanes == 2, s1, 0.0)
           + jnp.where(lanes == 3, s2, 0.0))
    o_ref[...] = out


HF = DF // 2


def _ffn_kernel(emap_ref, valid_ref, x_ref, w1_ref, b1_ref, w2_ref, b2_ref,
                wrow_ref, o_ref):
    t = pl.program_id(0)
    f = pl.program_id(1)

    @pl.when(valid_ref[t] != 0)
    def _():
        x = x_ref[...]
        h = jax.lax.dot_general(x, w1_ref[0], (((1,), (1,)), ((), ())),
                                preferred_element_type=jnp.float32)
        h = jax.nn.gelu(h + b1_ref[0])
        y = jax.lax.dot_general(h, w2_ref[0], (((1,), (1,)), ((), ())),
                                preferred_element_type=jnp.float32)
        w = wrow_ref[...]

        @pl.when(f == 0)
        def _():
            o_ref[...] = (y + b2_ref[0]) * w

        @pl.when(f != 0)
        def _():
            o_ref[...] += y * w


def kernel(moe_inp, gate_w, gate_b, w1, b1, w2, b2):
    gw_pad = jnp.zeros((EPAD, DM), jnp.float32).at[:NE].set(gate_w)
    gb_pad = jnp.full((1, EPAD), -1e30, jnp.float32).at[0, :NE].set(gate_b)
    gate_out = pl.pallas_call(
        _gate_kernel,
        out_shape=jax.ShapeDtypeStruct((NT, EPAD), jnp.float32),
    )(moe_inp, gw_pad, gb_pad)
    top_i = gate_out[:, :TK].astype(jnp.int32)
    gate_s = gate_out[:, TK:2 * TK]

    eflat = top_i.reshape(-1)
    wflat = gate_s.reshape(-1)
    tok = jnp.arange(NPAIR, dtype=jnp.int32) // TK
    onehot = (eflat[:, None] == jnp.arange(NE, dtype=jnp.int32)[None, :])
    cum = jnp.cumsum(onehot.astype(jnp.int32), axis=0)
    rank = jnp.sum(jnp.where(onehot, cum, 0), axis=1) - 1
    counts = cum[-1]
    nblk = (counts + BLK - 1) // BLK
    blk_end = jnp.cumsum(nblk)
    astart = (blk_end - nblk) * BLK
    pos = astart[eflat] + rank
    gidx = jnp.zeros((PROWS,), jnp.int32).at[pos].set(tok)
    x_pad = jnp.take(moe_inp, gidx, axis=0)
    w_pad = jnp.zeros((PROWS, 1), jnp.float32).at[pos, 0].set(wflat)
    tgrid = jnp.arange(TG, dtype=jnp.int32)
    emap = jnp.minimum(jnp.searchsorted(blk_end, tgrid, side='right'),
                       NE - 1).astype(jnp.int32)
    valid = (tgrid < blk_end[-1]).astype(jnp.int32)

    grid_spec = pltpu.PrefetchScalarGridSpec(
        num_scalar_prefetch=2,
        grid=(TG, 2),
        in_specs=[
            pl.BlockSpec((BLK, DM), lambda t, f, em, va: (t, 0)),
            pl.BlockSpec((1, HF, DM), lambda t, f, em, va: (em[t], f, 0)),
            pl.BlockSpec((1, 1, HF), lambda t, f, em, va: (em[t], 0, f)),
            pl.BlockSpec((1, DM, HF), lambda t, f, em, va: (em[t], 0, f)),
            pl.BlockSpec((1, 1, DM), lambda t, f, em, va: (em[t], 0, 0)),
            pl.BlockSpec((BLK, 1), lambda t, f, em, va: (t, 0)),
        ],
        out_specs=pl.BlockSpec((BLK, DM), lambda t, f, em, va: (t, 0)),
    )
    ysum = pl.pallas_call(
        _ffn_kernel,
        grid_spec=grid_spec,
        out_shape=jax.ShapeDtypeStruct((PROWS, DM), jnp.float32),
    )(emap, valid, x_pad, w1, b1.reshape(NE, 1, DF),
      w2, b2.reshape(NE, 1, DM), w_pad)

    pp = pos.reshape(NT, TK)
    out = jnp.take(ysum, pp[:, 0], axis=0) + jnp.take(ysum, pp[:, 1], axis=0)
    return out


# speedup vs baseline: 1.3681x; 1.0620x over previous
import jax
import jax.numpy as jnp
from jax.experimental import pallas as pl
from jax.experimental.pallas import tpu as pltpu

NE = 8
DM = 1024
DF = 4096
TK = 2
NT = 2048
EPAD = 128

BLK = 512
NPAIR = NT * TK
TG = NPAIR // BLK + NE
PROWS = TG * BLK


def _gate_kernel(x_ref, gw_ref, gb_ref, o_ref):
    logits = jax.lax.dot_general(
        x_ref[...], gw_ref[...], (((1,), (1,)), ((), ())),
        preferred_element_type=jnp.float32)
    logits = logits + gb_ref[...]
    lanes = jax.lax.broadcasted_iota(jnp.int32, (NT, EPAD), 1)
    m1 = jnp.max(logits, axis=1, keepdims=True)
    i1 = jnp.min(jnp.where(logits == m1, lanes, EPAD), axis=1, keepdims=True)
    l2 = jnp.where(lanes == i1, -jnp.inf, logits)
    m2 = jnp.max(l2, axis=1, keepdims=True)
    i2 = jnp.min(jnp.where(l2 == m2, lanes, EPAD), axis=1, keepdims=True)
    s1 = 1.0 / (1.0 + jnp.exp(m2 - m1))
    s2 = 1.0 - s1
    out = (jnp.where(lanes == 0, i1.astype(jnp.float32), 0.0)
           + jnp.where(lanes == 1, i2.astype(jnp.float32), 0.0)
           + jnp.where(lanes == 2, s1, 0.0)
           + jnp.where(lanes == 3, s2, 0.0))
    o_ref[...] = out


HF = DF // 2


def _ffn_kernel(emap_ref, valid_ref, x_ref, w1_ref, b1_ref, w2_ref, b2_ref,
                wrow_ref, o_ref):
    t = pl.program_id(0)
    f = pl.program_id(1)

    @pl.when(valid_ref[t] != 0)
    def _():
        x = x_ref[...]
        h = jax.lax.dot_general(x, w1_ref[0], (((1,), (1,)), ((), ())),
                                preferred_element_type=jnp.float32)
        h = jax.nn.gelu(h + b1_ref[0])
        y = jax.lax.dot_general(h, w2_ref[0], (((1,), (1,)), ((), ())),
                                preferred_element_type=jnp.float32)
        w = wrow_ref[...]

        @pl.when(f == 0)
        def _():
            o_ref[...] = (y + b2_ref[0]) * w

        @pl.when(f != 0)
        def _():
            o_ref[...] += y * w


def kernel(moe_inp, gate_w, gate_b, w1, b1, w2, b2):
    gw_pad = jnp.zeros((EPAD, DM), jnp.float32).at[:NE].set(gate_w)
    gb_pad = jnp.full((1, EPAD), -1e30, jnp.float32).at[0, :NE].set(gate_b)
    gate_out = pl.pallas_call(
        _gate_kernel,
        out_shape=jax.ShapeDtypeStruct((NT, EPAD), jnp.float32),
    )(moe_inp, gw_pad, gb_pad)
    top_i = gate_out[:, :TK].astype(jnp.int32)
    gate_s = gate_out[:, TK:2 * TK]

    eflat = top_i.reshape(-1)
    wflat = gate_s.reshape(-1)
    tok = jnp.arange(NPAIR, dtype=jnp.int32) // TK
    onehot = (eflat[:, None] == jnp.arange(NE, dtype=jnp.int32)[None, :])
    cum = jnp.cumsum(onehot.astype(jnp.int32), axis=0)
    rank = jnp.sum(jnp.where(onehot, cum, 0), axis=1) - 1
    counts = cum[-1]
    nblk = (counts + BLK - 1) // BLK
    blk_end = jnp.cumsum(nblk)
    astart = (blk_end - nblk) * BLK
    pos = astart[eflat] + rank
    gidx = jnp.zeros((PROWS,), jnp.int32).at[pos].set(tok)
    x_pad = jnp.take(moe_inp, gidx, axis=0)
    w_pad = jnp.zeros((PROWS, 1), jnp.float32).at[pos, 0].set(wflat)
    tgrid = jnp.arange(TG, dtype=jnp.int32)
    emap = jnp.minimum(jnp.searchsorted(blk_end, tgrid, side='right'),
                       NE - 1).astype(jnp.int32)
    valid = (tgrid < blk_end[-1]).astype(jnp.int32)

    grid_spec = pltpu.PrefetchScalarGridSpec(
        num_scalar_prefetch=2,
        grid=(TG, 2),
        in_specs=[
            pl.BlockSpec((BLK, DM), lambda t, f, em, va: (t, 0)),
            pl.BlockSpec((1, HF, DM),
                         lambda t, f, em, va: (em[t], (f + t) % 2, 0)),
            pl.BlockSpec((1, 1, HF),
                         lambda t, f, em, va: (em[t], 0, (f + t) % 2)),
            pl.BlockSpec((1, DM, HF),
                         lambda t, f, em, va: (em[t], 0, (f + t) % 2)),
            pl.BlockSpec((1, 1, DM), lambda t, f, em, va: (em[t], 0, 0)),
            pl.BlockSpec((BLK, 1), lambda t, f, em, va: (t, 0)),
        ],
        out_specs=pl.BlockSpec((BLK, DM), lambda t, f, em, va: (t, 0)),
    )
    ysum = pl.pallas_call(
        _ffn_kernel,
        grid_spec=grid_spec,
        out_shape=jax.ShapeDtypeStruct((PROWS, DM), jnp.float32),
    )(emap, valid, x_pad, w1, b1.reshape(NE, 1, DF),
      w2, b2.reshape(NE, 1, DM), w_pad)

    pp = pos.reshape(NT, TK)
    out = jnp.take(ysum, pp[:, 0], axis=0) + jnp.take(ysum, pp[:, 1], axis=0)
    return out
